# baseline (device time: 68196 ns/iter reference)
import jax
import jax.numpy as jnp
from jax import lax
from jax.experimental import pallas as pl
from jax.experimental.pallas import tpu as pltpu

N_DEV = 4
B, SQ, SKV = 2, 512, 512
H_LOCAL, DH = 8, 64
D_HEADS = H_LOCAL * DH
D_MODEL = 768


def kernel(x, Wq, K_ext, V_ext, Wo):
    K2 = K_ext.reshape(B, SKV, D_HEADS)
    V2 = V_ext.reshape(B, SKV, D_HEADS)

    def body(x_ref, wq_ref, k_ref, v_ref, wo_ref, out_ref,
             comm_ref, send_sems, recv_sems):
        my = lax.axis_index("i")
        left = lax.rem(my + N_DEV - 1, N_DEV)
        right = lax.rem(my + 1, N_DEV)

        barrier_sem = pltpu.get_barrier_semaphore()
        for nbr in (left, right):
            pl.semaphore_signal(
                barrier_sem, inc=1,
                device_id=(nbr,), device_id_type=pl.DeviceIdType.MESH,
            )
        pl.semaphore_wait(barrier_sem, 2)

        qi = lax.broadcasted_iota(jnp.int32, (SQ, SKV), 0)
        ki = lax.broadcasted_iota(jnp.int32, (SQ, SKV), 1)
        mask = (jnp.abs(qi - ki) <= 128) | (ki < 32) | (qi < 32)

        wq_slice = wq_ref[:, pl.ds(my * D_HEADS, D_HEADS)].astype(jnp.bfloat16)
        for b in range(B):
            xb = x_ref[b, :, :].astype(jnp.bfloat16)
            qb = jnp.dot(xb, wq_slice,
                         preferred_element_type=jnp.float32)
            for h in range(H_LOCAL):
                sl = slice(h * DH, (h + 1) * DH)
                qh = qb[:, sl].astype(jnp.bfloat16)
                kh = k_ref[b, :, sl].astype(jnp.bfloat16)
                vh = v_ref[b, :, sl].astype(jnp.bfloat16)
                s = lax.dot_general(
                    qh, kh, (((1,), (1,)), ((), ())),
                    preferred_element_type=jnp.float32) * 0.125
                s = jnp.where(mask, s, -1e9)
                m = jnp.max(s, axis=-1, keepdims=True)
                w = jnp.exp(s - m)
                w = w / jnp.sum(w, axis=-1, keepdims=True)
                ctx = jnp.dot(w.astype(jnp.bfloat16), vh,
                              preferred_element_type=jnp.float32)
                comm_ref[0, b, :, sl] = ctx.astype(jnp.bfloat16)

        def accum(slot, origin, init):
            wo_sl = wo_ref[pl.ds(origin * D_HEADS, D_HEADS), :].astype(
                jnp.bfloat16)
            for b in range(B):
                part = jnp.dot(comm_ref[slot, b, :, :], wo_sl,
                               preferred_element_type=jnp.float32)
                if init:
                    out_ref[b, :, :] = part
                else:
                    out_ref[b, :, :] = out_ref[b, :, :] + part

        accum(0, my, init=True)

        for hop in range(N_DEV - 1):
            rdma = pltpu.make_async_remote_copy(
                src_ref=comm_ref.at[hop],
                dst_ref=comm_ref.at[hop + 1],
                send_sem=send_sems.at[hop],
                recv_sem=recv_sems.at[hop + 1],
                device_id=(right,),
                device_id_type=pl.DeviceIdType.MESH,
            )
            rdma.start()
            rdma.wait()
            origin = lax.rem(my + N_DEV - 1 - hop, N_DEV)
            accum(hop + 1, origin, init=False)

    return pl.pallas_call(
        body,
        out_shape=jax.ShapeDtypeStruct((B, SQ, D_MODEL), jnp.float32),
        in_specs=[pl.BlockSpec(memory_space=pltpu.VMEM)] * 5,
        out_specs=pl.BlockSpec(memory_space=pltpu.VMEM),
        scratch_shapes=[
            pltpu.VMEM((N_DEV, B, SQ, D_HEADS), jnp.bfloat16),
            pltpu.SemaphoreType.DMA((N_DEV,)),
            pltpu.SemaphoreType.DMA((N_DEV,)),
        ],
        compiler_params=pltpu.CompilerParams(collective_id=0),
    )(x, Wq, K2, V2, Wo)


# device time: 43199 ns/iter; 1.5786x vs baseline; 1.5786x over previous
import jax
import jax.numpy as jnp
from jax import lax
from jax.experimental import pallas as pl
from jax.experimental.pallas import tpu as pltpu

N_DEV = 4
B, SQ, SKV = 2, 512, 512
H_LOCAL, DH = 8, 64
D_HEADS = H_LOCAL * DH
D_MODEL = 768
N_HOP = N_DEV - 1


def kernel(x, Wq, K_ext, V_ext, Wo):
    K2 = K_ext.reshape(B, SKV, D_HEADS)
    V2 = V_ext.reshape(B, SKV, D_HEADS)

    def body(x_ref, wq_ref, k_ref, v_ref, wo_ref, out_ref,
             cw_ref, ccw_ref, scw, rcw, sccw, rccw):
        my = lax.axis_index("i")
        left = lax.rem(my + N_DEV - 1, N_DEV)
        right = lax.rem(my + 1, N_DEV)

        barrier_sem = pltpu.get_barrier_semaphore()
        for nbr in (left, right):
            pl.semaphore_signal(
                barrier_sem, inc=1,
                device_id=(nbr,), device_id_type=pl.DeviceIdType.MESH,
            )
        pl.semaphore_wait(barrier_sem, 2)

        qi = lax.broadcasted_iota(jnp.int32, (SQ, SKV), 0)
        ki = lax.broadcasted_iota(jnp.int32, (SQ, SKV), 1)
        mask = (jnp.abs(qi - ki) <= 128) | (ki < 32) | (qi < 32)

        wq_slice = (wq_ref[:, pl.ds(my * D_HEADS, D_HEADS)] * 0.125).astype(
            jnp.bfloat16)

        def attn_batch(b, dst_ref):
            xb = x_ref[b, :, :].astype(jnp.bfloat16)
            qb = jnp.dot(xb, wq_slice,
                         preferred_element_type=jnp.float32)
            for h in range(H_LOCAL):
                sl = slice(h * DH, (h + 1) * DH)
                qh = qb[:, sl].astype(jnp.bfloat16)
                kh = k_ref[b, :, sl].astype(jnp.bfloat16)
                vh = v_ref[b, :, sl].astype(jnp.bfloat16)
                s = lax.dot_general(
                    qh, kh, (((1,), (1,)), ((), ())),
                    preferred_element_type=jnp.float32)
                w = jnp.where(mask, jnp.exp(s), 0.0)
                denom = jnp.sum(w, axis=-1, keepdims=True)
                ctx = jnp.dot(w.astype(jnp.bfloat16), vh,
                              preferred_element_type=jnp.float32) / denom
                dst_ref[:, sl] = ctx.astype(jnp.bfloat16)

        def mk(buf, ssem, rsem, h, target):
            return pltpu.make_async_remote_copy(
                src_ref=buf.at[h],
                dst_ref=buf.at[h + 1],
                send_sem=ssem.at[h],
                recv_sem=rsem.at[h],
                device_id=(target,),
                device_id_type=pl.DeviceIdType.MESH,
            )

        cw = [mk(cw_ref, scw, rcw, h, right) for h in range(N_HOP)]
        ccw = [mk(ccw_ref, sccw, rccw, h, left) for h in range(N_HOP)]

        attn_batch(0, cw_ref.at[0])
        cw[0].start()
        attn_batch(1, ccw_ref.at[0])
        ccw[0].start()

        wo_my = wo_ref[pl.ds(my * D_HEADS, D_HEADS), :].astype(jnp.bfloat16)
        out_ref[0, :, :] = jnp.dot(cw_ref[0, :, :], wo_my,
                                   preferred_element_type=jnp.float32)
        out_ref[1, :, :] = jnp.dot(ccw_ref[0, :, :], wo_my,
                                   preferred_element_type=jnp.float32)

        for hop in range(N_HOP):
            cw[hop].wait_recv()
            if hop + 1 < N_HOP:
                cw[hop + 1].start()
            ccw[hop].wait_recv()
            if hop + 1 < N_HOP:
                ccw[hop + 1].start()

            slot = hop + 1
            o_cw = lax.rem(my + N_DEV - 1 - hop, N_DEV)
            o_ccw = lax.rem(my + 1 + hop, N_DEV)
            wo_cw = wo_ref[pl.ds(o_cw * D_HEADS, D_HEADS), :].astype(
                jnp.bfloat16)
            out_ref[0, :, :] = out_ref[0, :, :] + jnp.dot(
                cw_ref[slot, :, :], wo_cw,
                preferred_element_type=jnp.float32)
            wo_ccw = wo_ref[pl.ds(o_ccw * D_HEADS, D_HEADS), :].astype(
                jnp.bfloat16)
            out_ref[1, :, :] = out_ref[1, :, :] + jnp.dot(
                ccw_ref[slot, :, :], wo_ccw,
                preferred_element_type=jnp.float32)

        for h in range(N_HOP):
            cw[h].wait_send()
            ccw[h].wait_send()

    return pl.pallas_call(
        body,
        out_shape=jax.ShapeDtypeStruct((B, SQ, D_MODEL), jnp.float32),
        in_specs=[pl.BlockSpec(memory_space=pltpu.VMEM)] * 5,
        out_specs=pl.BlockSpec(memory_space=pltpu.VMEM),
        scratch_shapes=[
            pltpu.VMEM((N_DEV, SQ, D_HEADS), jnp.bfloat16),
            pltpu.VMEM((N_DEV, SQ, D_HEADS), jnp.bfloat16),
            pltpu.SemaphoreType.DMA((N_HOP,)),
            pltpu.SemaphoreType.DMA((N_HOP,)),
            pltpu.SemaphoreType.DMA((N_HOP,)),
            pltpu.SemaphoreType.DMA((N_HOP,)),
        ],
        compiler_params=pltpu.CompilerParams(collective_id=0),
    )(x, Wq, K2, V2, Wo)


# device time: 35752 ns/iter; 1.9075x vs baseline; 1.2083x over previous
import jax
import jax.numpy as jnp
from jax import lax
from jax.experimental import pallas as pl
from jax.experimental.pallas import tpu as pltpu

N_DEV = 4
B, SQ, SKV = 2, 512, 512
H_LOCAL, DH = 8, 64
D_HEADS = H_LOCAL * DH
D_MODEL = 768
NG = 4
GW = D_HEADS // NG

CW1_B0, CCW1_B0, CW2_B0, CCW1_B1, CW1_B1, CCW2_B1 = range(6)


def kernel(x, Wq, K_ext, V_ext, Wo):
    K2 = K_ext.reshape(B, SKV, D_HEADS)
    V2 = V_ext.reshape(B, SKV, D_HEADS)

    def body(x_ref, wq_ref, k_ref, v_ref, wo_ref, out_ref,
             mine_ref, recv_ref, send_sems, recv_sems):
        my = lax.axis_index("i")
        left = lax.rem(my + N_DEV - 1, N_DEV)
        right = lax.rem(my + 1, N_DEV)
        opp = lax.rem(my + 2, N_DEV)

        barrier_sem = pltpu.get_barrier_semaphore()
        for nbr in (left, right):
            pl.semaphore_signal(
                barrier_sem, inc=1,
                device_id=(nbr,), device_id_type=pl.DeviceIdType.MESH,
            )
        pl.semaphore_wait(barrier_sem, 2)

        qi = lax.broadcasted_iota(jnp.int32, (SQ, SKV), 0)
        ki = lax.broadcasted_iota(jnp.int32, (SQ, SKV), 1)
        mask = (jnp.abs(qi - ki) <= 128) | (ki < 32) | (qi < 32)

        wq_slice = (wq_ref[:, pl.ds(my * D_HEADS, D_HEADS)] * 0.125).astype(
            jnp.bfloat16)

        def attn_head(qb, b, h):
            sl = slice(h * DH, (h + 1) * DH)
            qh = qb[:, sl].astype(jnp.bfloat16)
            kh = k_ref[b, :, sl].astype(jnp.bfloat16)
            vh = v_ref[b, :, sl].astype(jnp.bfloat16)
            s = lax.dot_general(
                qh, kh, (((1,), (1,)), ((), ())),
                preferred_element_type=jnp.float32)
            w = jnp.where(mask, jnp.exp(s), 0.0)
            denom = jnp.sum(w, axis=-1, keepdims=True)
            ctx = jnp.dot(w.astype(jnp.bfloat16), vh,
                          preferred_element_type=jnp.float32) / denom
            mine_ref[b, :, sl] = ctx.astype(jnp.bfloat16)

        def gsl(g):
            return pl.ds(g * GW, GW)

        def mk(src, dst_row, row, g, target):
            return pltpu.make_async_remote_copy(
                src_ref=src.at[:, gsl(g)],
                dst_ref=recv_ref.at[dst_row, :, gsl(g)],
                send_sem=send_sems.at[row, g],
                recv_sem=recv_sems.at[row, g],
                device_id=(target,),
                device_id_type=pl.DeviceIdType.MESH,
            )

        cw1_b0 = [mk(mine_ref.at[0], CW1_B0, CW1_B0, g, right)
                  for g in range(NG)]
        ccw1_b0 = [mk(mine_ref.at[0], CCW1_B0, CCW1_B0, g, left)
                   for g in range(NG)]
        cw2_b0 = [mk(recv_ref.at[CW1_B0], CW2_B0, CW2_B0, g, right)
                  for g in range(NG)]
        ccw1_b1 = [mk(mine_ref.at[1], CCW1_B1, CCW1_B1, g, left)
                   for g in range(NG)]
        cw1_b1 = [mk(mine_ref.at[1], CW1_B1, CW1_B1, g, right)
                  for g in range(NG)]
        ccw2_b1 = [mk(recv_ref.at[CCW1_B1], CCW2_B1, CCW2_B1, g, left)
                   for g in range(NG)]

        qb0 = jnp.dot(x_ref[0, :, :].astype(jnp.bfloat16), wq_slice,
                      preferred_element_type=jnp.float32)
        for g in range(NG):
            attn_head(qb0, 0, 2 * g)
            attn_head(qb0, 0, 2 * g + 1)
            cw1_b0[g].start()
            ccw1_b0[g].start()

        qb1 = jnp.dot(x_ref[1, :, :].astype(jnp.bfloat16), wq_slice,
                      preferred_element_type=jnp.float32)
        for g in range(NG):
            attn_head(qb1, 1, 2 * g)
            attn_head(qb1, 1, 2 * g + 1)
            ccw1_b1[g].start()
            cw1_b1[g].start()
            cw1_b0[g].wait_recv()
            cw2_b0[g].start()

        wo_my = wo_ref[pl.ds(my * D_HEADS, D_HEADS), :].astype(jnp.bfloat16)
        out_ref[0, :, :] = jnp.dot(mine_ref[0, :, :], wo_my,
                                   preferred_element_type=jnp.float32)
        out_ref[1, :, :] = jnp.dot(mine_ref[1, :, :], wo_my,
                                   preferred_element_type=jnp.float32)

        for g in range(NG):
            ccw1_b1[g].wait_recv()
            ccw2_b1[g].start()

        def accum(b, row, origin):
            wo_sl = wo_ref[pl.ds(origin * D_HEADS, D_HEADS), :].astype(
                jnp.bfloat16)
            out_ref[b, :, :] = out_ref[b, :, :] + jnp.dot(
                recv_ref[row, :, :], wo_sl,
                preferred_element_type=jnp.float32)

        accum(0, CW1_B0, left)
        accum(1, CCW1_B1, right)
        for g in range(NG):
            ccw1_b0[g].wait_recv()
        accum(0, CCW1_B0, right)
        for g in range(NG):
            cw1_b1[g].wait_recv()
        accum(1, CW1_B1, left)
        for g in range(NG):
            cw2_b0[g].wait_recv()
        accum(0, CW2_B0, opp)
        for g in range(NG):
            ccw2_b1[g].wait_recv()
        accum(1, CCW2_B1, opp)

        for g in range(NG):
            for d in (cw1_b0, ccw1_b0, cw2_b0, ccw1_b1, cw1_b1, ccw2_b1):
                d[g].wait_send()

    return pl.pallas_call(
        body,
        out_shape=jax.ShapeDtypeStruct((B, SQ, D_MODEL), jnp.float32),
        in_specs=[pl.BlockSpec(memory_space=pltpu.VMEM)] * 5,
        out_specs=pl.BlockSpec(memory_space=pltpu.VMEM),
        scratch_shapes=[
            pltpu.VMEM((B, SQ, D_HEADS), jnp.bfloat16),
            pltpu.VMEM((6, SQ, D_HEADS), jnp.bfloat16),
            pltpu.SemaphoreType.DMA((6, NG)),
            pltpu.SemaphoreType.DMA((6, NG)),
        ],
        compiler_params=pltpu.CompilerParams(collective_id=0),
    )(x, Wq, K2, V2, Wo)
